# baseline (speedup 1.0000x reference)
import functools

import jax
import jax.numpy as jnp
import numpy as np
from jax import lax
from jax.experimental import pallas as pl
from jax.experimental.pallas import tpu as pltpu
from jax.experimental.pallas import tpu_sc as plsc

VOCAB = 100000
MAX_LEN = 200
EMBED = 64
B = 1024
L = 200

NUM_CORES = 2
NUM_SUBCORES = 16
NUM_WORKERS = NUM_CORES * NUM_SUBCORES
PHASES = EMBED // NUM_WORKERS
LANES = 16
LG = 8
NLG = L // LG


def _positional_encoding(length, depth):
    depth = depth / 2
    positions = np.arange(length)[:, np.newaxis]
    depths = np.arange(depth)[np.newaxis, :] / depth
    angle_rates = 1 / 10000.0 ** depths
    angle_rads = positions * angle_rates
    enc = np.concatenate([np.sin(angle_rads), np.cos(angle_rads)], axis=-1)
    return enc.astype(np.float32)


_POS = _positional_encoding(MAX_LEN, EMBED)
_POS_B = np.repeat(_POS.T[:, :, None], LANES, axis=2).reshape(EMBED, L * LANES)
_SCALE = float(np.sqrt(EMBED))


def _sc_body(xt_hbm, posb_hbm, tt_hbm, out_hbm,
             chan_v, pos_v, ibufs, obufs, isems, osems):
    c = lax.axis_index("c")
    s = lax.axis_index("s")
    wid = s * NUM_CORES + c

    for phase in range(PHASES):
        e = phase * NUM_WORKERS + wid
        pltpu.sync_copy(tt_hbm.at[e], chan_v)
        pltpu.sync_copy(posb_hbm.at[e], pos_v)

        def islab(lg):
            return xt_hbm.at[pl.ds(lg * LG, LG)]

        for bi in range(2):
            pltpu.async_copy(islab(bi), ibufs[bi], isems[bi])

        def do_slab(lg0, blg):
            lg = lg0 + blg
            pltpu.make_async_copy(islab(lg), ibufs[blg], isems[blg]).wait()
            def wait_out(p):
                pltpu.make_async_copy(obufs[p], out_hbm.at[0, 0],
                                      osems[p]).wait()

            for dl in range(LG):
                l = lg * LG + dl
                p = dl % 2
                if dl >= 2 or phase > 0:
                    wait_out(p)
                else:
                    @pl.when(lg > 0)
                    def _():
                        wait_out(p)

                posv = pos_v[pl.ds(l * LANES, LANES)]

                @pl.loop(0, B // LANES, unroll=8)
                def _vec(j):
                    sl = pl.ds(j * LANES, LANES)
                    idxv = ibufs[blg][dl, sl]
                    g = plsc.load_gather(chan_v, [idxv])
                    obufs[p][sl] = g * _SCALE + posv

                pltpu.async_copy(obufs[p], out_hbm.at[l, e], osems[p])

            @pl.when(lg + 2 < NLG)
            def _():
                pltpu.async_copy(islab(lg + 2), ibufs[blg], isems[blg])

        @pl.loop(0, NLG - 1, step=2)
        def _slabs(lg0):
            for blg in range(2):
                do_slab(lg0, blg)

        do_slab(NLG - 1, 0)

    for p in range(2):
        pltpu.make_async_copy(obufs[p], out_hbm.at[0, 0], osems[p]).wait()


@functools.partial(
    pl.kernel,
    out_type=jax.ShapeDtypeStruct((L, EMBED, B), jnp.float32),
    mesh=plsc.VectorSubcoreMesh(core_axis_name="c", subcore_axis_name="s"),
    compiler_params=pltpu.CompilerParams(use_tc_tiling_on_sc=True,
                                         needs_layout_passes=False),
    scratch_types=[
        pltpu.VMEM((VOCAB,), jnp.float32),
        pltpu.VMEM((L * LANES,), jnp.float32),
        [pltpu.VMEM((LG, B), jnp.int32) for _ in range(2)],
        [pltpu.VMEM((B,), jnp.float32) for _ in range(2)],
        [pltpu.SemaphoreType.DMA for _ in range(2)],
        [pltpu.SemaphoreType.DMA for _ in range(2)],
    ],
)
def _sc_embed(xt_hbm, posb_hbm, tt_hbm, out_hbm,
              chan_v, pos_v, ibufs, obufs, isems, osems):
    _sc_body(xt_hbm, posb_hbm, tt_hbm, out_hbm,
             chan_v, pos_v, ibufs, obufs, isems, osems)


def kernel(x, table):
    posb = jnp.asarray(_POS_B)
    out3 = _sc_embed(x.T, posb, table.T)
    return out3.transpose(2, 0, 1)


# speedup vs baseline: 2.9917x; 2.9917x over previous
import functools

import jax
import jax.numpy as jnp
import numpy as np
from jax import lax
from jax.experimental import pallas as pl
from jax.experimental.pallas import tpu as pltpu
from jax.experimental.pallas import tpu_sc as plsc

VOCAB = 100000
MAX_LEN = 200
EMBED = 64
B = 1024
L = 200

NUM_CORES = 2
NUM_SUBCORES = 16
NUM_WORKERS = NUM_CORES * NUM_SUBCORES
PHASES = EMBED // NUM_WORKERS
LANES = 16
LG = 8
NLG = L // LG


def _positional_encoding(length, depth):
    depth = depth / 2
    positions = np.arange(length)[:, np.newaxis]
    depths = np.arange(depth)[np.newaxis, :] / depth
    angle_rates = 1 / 10000.0 ** depths
    angle_rads = positions * angle_rates
    enc = np.concatenate([np.sin(angle_rads), np.cos(angle_rads)], axis=-1)
    return enc.astype(np.float32)


_POS = _positional_encoding(MAX_LEN, EMBED)
_POS_B = np.repeat(_POS.T[:, :, None], LANES, axis=2).reshape(EMBED, L * LANES)
_SCALE = float(np.sqrt(EMBED))


def _sc_body(xt_hbm, posb_hbm, tt_hbm, out_hbm,
             chan_v, pos_v, ibufs, obufs, isems, osems):
    c = lax.axis_index("c")
    s = lax.axis_index("s")
    wid = s * NUM_CORES + c

    for phase in range(PHASES):
        e = phase * NUM_WORKERS + wid
        pltpu.sync_copy(tt_hbm.at[e], chan_v)
        pltpu.sync_copy(posb_hbm.at[e], pos_v)

        def islab(lg):
            return xt_hbm.at[pl.ds(lg * LG, LG)]

        for bi in range(2):
            pltpu.async_copy(islab(bi), ibufs[bi], isems[bi])

        def do_slab(lg0, blg):
            lg = lg0 + blg
            pltpu.make_async_copy(islab(lg), ibufs[blg], isems[blg]).wait()
            def wait_out(p):
                pltpu.make_async_copy(obufs[p], out_hbm.at[0, 0],
                                      osems[p]).wait()

            for dl in range(LG):
                l = lg * LG + dl
                p = dl % 2
                if dl >= 2 or phase > 0:
                    wait_out(p)
                else:
                    @pl.when(lg > 0)
                    def _():
                        wait_out(p)

                posv = pos_v[pl.ds(l * LANES, LANES)]

                @plsc.parallel_loop(0, B // LANES, unroll=8)
                def _vec(j):
                    sl = pl.ds(j * LANES, LANES)
                    idxv = ibufs[blg][dl, sl]
                    g = plsc.load_gather(chan_v, [idxv])
                    obufs[p][sl] = g * _SCALE + posv

                pltpu.async_copy(obufs[p], out_hbm.at[l, e], osems[p])

            @pl.when(lg + 2 < NLG)
            def _():
                pltpu.async_copy(islab(lg + 2), ibufs[blg], isems[blg])

        @pl.loop(0, NLG - 1, step=2)
        def _slabs(lg0):
            for blg in range(2):
                do_slab(lg0, blg)

        do_slab(NLG - 1, 0)

    for p in range(2):
        pltpu.make_async_copy(obufs[p], out_hbm.at[0, 0], osems[p]).wait()


@functools.partial(
    pl.kernel,
    out_type=jax.ShapeDtypeStruct((L, EMBED, B), jnp.float32),
    mesh=plsc.VectorSubcoreMesh(core_axis_name="c", subcore_axis_name="s"),
    compiler_params=pltpu.CompilerParams(use_tc_tiling_on_sc=True,
                                         needs_layout_passes=False),
    scratch_types=[
        pltpu.VMEM((VOCAB,), jnp.float32),
        pltpu.VMEM((L * LANES,), jnp.float32),
        [pltpu.VMEM((LG, B), jnp.int32) for _ in range(2)],
        [pltpu.VMEM((B,), jnp.float32) for _ in range(2)],
        [pltpu.SemaphoreType.DMA for _ in range(2)],
        [pltpu.SemaphoreType.DMA for _ in range(2)],
    ],
)
def _sc_embed(xt_hbm, posb_hbm, tt_hbm, out_hbm,
              chan_v, pos_v, ibufs, obufs, isems, osems):
    _sc_body(xt_hbm, posb_hbm, tt_hbm, out_hbm,
             chan_v, pos_v, ibufs, obufs, isems, osems)


def kernel(x, table):
    posb = jnp.asarray(_POS_B)
    out3 = _sc_embed(x.T, posb, table.T)
    return out3.transpose(2, 0, 1)


# speedup vs baseline: 4.3134x; 1.4418x over previous
import functools

import jax
import jax.numpy as jnp
import numpy as np
from jax import lax
from jax.experimental import pallas as pl
from jax.experimental.pallas import tpu as pltpu
from jax.experimental.pallas import tpu_sc as plsc

VOCAB = 100000
MAX_LEN = 200
EMBED = 64
B = 1024
L = 200

NUM_CORES = 2
NUM_SUBCORES = 16
NUM_WORKERS = NUM_CORES * NUM_SUBCORES
PHASES = EMBED // NUM_WORKERS
LANES = 16
LG = 4
NLG = L // LG


def _positional_encoding(length, depth):
    depth = depth / 2
    positions = np.arange(length)[:, np.newaxis]
    depths = np.arange(depth)[np.newaxis, :] / depth
    angle_rates = 1 / 10000.0 ** depths
    angle_rads = positions * angle_rates
    enc = np.concatenate([np.sin(angle_rads), np.cos(angle_rads)], axis=-1)
    return enc.astype(np.float32)


_POS = _positional_encoding(MAX_LEN, EMBED)
_POS_B = np.repeat(_POS.T[:, :, None], LANES, axis=2).reshape(EMBED, L * LANES)
_SCALE = float(np.sqrt(EMBED))


def _sc_body(xt_hbm, posb_hbm, tt_hbm, out_hbm,
             chan_v, pos_v, xsh, ibufs, obufs, isems, osems):
    c = lax.axis_index("c")
    s = lax.axis_index("s")
    wid = s * NUM_CORES + c

    @pl.when(s == 0)
    def _():
        pltpu.sync_copy(xt_hbm, xsh)

    plsc.subcore_barrier()

    for phase in range(PHASES):
        e = phase * NUM_WORKERS + wid
        pltpu.sync_copy(tt_hbm.at[e], chan_v)
        pltpu.sync_copy(posb_hbm.at[e], pos_v)

        def islab(lg):
            return xsh.at[pl.ds(lg * LG, LG)]

        for bi in range(2):
            pltpu.async_copy(islab(bi), ibufs[bi], isems[bi])

        def do_slab(lg0, blg):
            lg = lg0 + blg
            pltpu.make_async_copy(islab(lg), ibufs[blg], isems[blg]).wait()
            def wait_out(p):
                pltpu.make_async_copy(obufs[p], out_hbm.at[0, 0],
                                      osems[p]).wait()

            for dl in range(LG):
                l = lg * LG + dl
                p = dl % 2
                if dl >= 2 or phase > 0:
                    wait_out(p)
                else:
                    @pl.when(lg > 0)
                    def _():
                        wait_out(p)

                posv = pos_v[pl.ds(l * LANES, LANES)]

                @plsc.parallel_loop(0, B // LANES, unroll=8)
                def _vec(j):
                    sl = pl.ds(j * LANES, LANES)
                    idxv = ibufs[blg][dl, sl]
                    g = plsc.load_gather(chan_v, [idxv])
                    obufs[p][sl] = g * _SCALE + posv

                pltpu.async_copy(obufs[p], out_hbm.at[l, e], osems[p])

            @pl.when(lg + 2 < NLG)
            def _():
                pltpu.async_copy(islab(lg + 2), ibufs[blg], isems[blg])

        @pl.loop(0, NLG, step=2)
        def _slabs(lg0):
            for blg in range(2):
                do_slab(lg0, blg)

    for p in range(2):
        pltpu.make_async_copy(obufs[p], out_hbm.at[0, 0], osems[p]).wait()


@functools.partial(
    pl.kernel,
    out_type=jax.ShapeDtypeStruct((L, EMBED, B), jnp.float32),
    mesh=plsc.VectorSubcoreMesh(core_axis_name="c", subcore_axis_name="s"),
    compiler_params=pltpu.CompilerParams(use_tc_tiling_on_sc=True,
                                         needs_layout_passes=False),
    scratch_types=[
        pltpu.VMEM((VOCAB,), jnp.float32),
        pltpu.VMEM((L * LANES,), jnp.float32),
        pltpu.VMEM_SHARED((L, B), jnp.int32),
        [pltpu.VMEM((LG, B), jnp.int32) for _ in range(2)],
        [pltpu.VMEM((B,), jnp.float32) for _ in range(2)],
        [pltpu.SemaphoreType.DMA for _ in range(2)],
        [pltpu.SemaphoreType.DMA for _ in range(2)],
    ],
)
def _sc_embed(xt_hbm, posb_hbm, tt_hbm, out_hbm,
              chan_v, pos_v, xsh, ibufs, obufs, isems, osems):
    _sc_body(xt_hbm, posb_hbm, tt_hbm, out_hbm,
             chan_v, pos_v, xsh, ibufs, obufs, isems, osems)


def kernel(x, table):
    posb = jnp.asarray(_POS_B)
    out3 = _sc_embed(x.T, posb, table.T)
    return out3.transpose(2, 0, 1)


# speedup vs baseline: 4.3232x; 1.0023x over previous
import functools

import jax
import jax.numpy as jnp
import numpy as np
from jax import lax
from jax.experimental import pallas as pl
from jax.experimental.pallas import tpu as pltpu
from jax.experimental.pallas import tpu_sc as plsc

VOCAB = 100000
MAX_LEN = 200
EMBED = 64
B = 1024
L = 200

NUM_CORES = 2
NUM_SUBCORES = 16
NUM_WORKERS = NUM_CORES * NUM_SUBCORES
PHASES = EMBED // NUM_WORKERS
LANES = 16
LG = 4
NLG = L // LG


def _positional_encoding(length, depth):
    depth = depth / 2
    positions = np.arange(length)[:, np.newaxis]
    depths = np.arange(depth)[np.newaxis, :] / depth
    angle_rates = 1 / 10000.0 ** depths
    angle_rads = positions * angle_rates
    enc = np.concatenate([np.sin(angle_rads), np.cos(angle_rads)], axis=-1)
    return enc.astype(np.float32)


_POS = _positional_encoding(MAX_LEN, EMBED)
_POS_B = np.repeat(_POS.T[:, :, None], LANES, axis=2).reshape(EMBED, L * LANES)
_SCALE = float(np.sqrt(EMBED))


def _sc_body(xt_hbm, posb_hbm, tt_hbm, out_hbm,
             chan_v, pos_v, xsh, ibufs, obufs, isems, osems):
    c = lax.axis_index("c")
    s = lax.axis_index("s")
    wid = s * NUM_CORES + c

    @pl.when(s == 0)
    def _():
        pltpu.sync_copy(xt_hbm, xsh)

    plsc.subcore_barrier()

    for phase in range(PHASES):
        e = phase * NUM_WORKERS + wid
        pltpu.sync_copy(tt_hbm.at[e], chan_v)
        pltpu.sync_copy(posb_hbm.at[e], pos_v)

        def islab(lg):
            return xsh.at[pl.ds(lg * LG, LG)]

        for bi in range(2):
            pltpu.async_copy(islab(bi), ibufs[bi], isems[bi])

        def do_slab(lg0, blg):
            lg = lg0 + blg
            pltpu.make_async_copy(islab(lg), ibufs[blg], isems[blg]).wait()
            def wait_out(p):
                pltpu.make_async_copy(obufs[p], out_hbm.at[0, 0],
                                      osems[p]).wait()

            for dl in range(LG):
                l = lg * LG + dl
                p = dl % 2
                if dl >= 2 or phase > 0:
                    wait_out(p)
                else:
                    @pl.when(lg > 0)
                    def _():
                        wait_out(p)

                posv = pos_v[pl.ds(l * LANES, LANES)]

                @plsc.parallel_loop(0, B // LANES, unroll=16)
                def _vec(j):
                    sl = pl.ds(j * LANES, LANES)
                    idxv = ibufs[blg][dl, sl]
                    g = plsc.load_gather(chan_v, [idxv])
                    obufs[p][sl] = g * _SCALE + posv

                pltpu.async_copy(obufs[p], out_hbm.at[l, e], osems[p])

            @pl.when(lg + 2 < NLG)
            def _():
                pltpu.async_copy(islab(lg + 2), ibufs[blg], isems[blg])

        @pl.loop(0, NLG, step=2)
        def _slabs(lg0):
            for blg in range(2):
                do_slab(lg0, blg)

    for p in range(2):
        pltpu.make_async_copy(obufs[p], out_hbm.at[0, 0], osems[p]).wait()


@functools.partial(
    pl.kernel,
    out_type=jax.ShapeDtypeStruct((L, EMBED, B), jnp.float32),
    mesh=plsc.VectorSubcoreMesh(core_axis_name="c", subcore_axis_name="s"),
    compiler_params=pltpu.CompilerParams(use_tc_tiling_on_sc=True,
                                         needs_layout_passes=False),
    scratch_types=[
        pltpu.VMEM((VOCAB,), jnp.float32),
        pltpu.VMEM((L * LANES,), jnp.float32),
        pltpu.VMEM_SHARED((L, B), jnp.int32),
        [pltpu.VMEM((LG, B), jnp.int32) for _ in range(2)],
        [pltpu.VMEM((B,), jnp.float32) for _ in range(2)],
        [pltpu.SemaphoreType.DMA for _ in range(2)],
        [pltpu.SemaphoreType.DMA for _ in range(2)],
    ],
)
def _sc_embed(xt_hbm, posb_hbm, tt_hbm, out_hbm,
              chan_v, pos_v, xsh, ibufs, obufs, isems, osems):
    _sc_body(xt_hbm, posb_hbm, tt_hbm, out_hbm,
             chan_v, pos_v, xsh, ibufs, obufs, isems, osems)


def kernel(x, table):
    posb = jnp.asarray(_POS_B)
    out3 = _sc_embed(x.T, posb, table.T)
    return out3.transpose(2, 0, 1)


# speedup vs baseline: 4.4313x; 1.0250x over previous
import functools

import jax
import jax.numpy as jnp
import numpy as np
from jax import lax
from jax.experimental import pallas as pl
from jax.experimental.pallas import tpu as pltpu
from jax.experimental.pallas import tpu_sc as plsc

VOCAB = 100000
MAX_LEN = 200
EMBED = 64
B = 1024
L = 200

NUM_CORES = 2
NUM_SUBCORES = 16
NUM_WORKERS = NUM_CORES * NUM_SUBCORES
PHASES = EMBED // NUM_WORKERS
LANES = 16
LG = 4
NLG = L // LG


def _positional_encoding(length, depth):
    depth = depth / 2
    positions = np.arange(length)[:, np.newaxis]
    depths = np.arange(depth)[np.newaxis, :] / depth
    angle_rates = 1 / 10000.0 ** depths
    angle_rads = positions * angle_rates
    enc = np.concatenate([np.sin(angle_rads), np.cos(angle_rads)], axis=-1)
    return enc.astype(np.float32)


_POS = _positional_encoding(MAX_LEN, EMBED)
_POS_B = np.ascontiguousarray(_POS.T)
_SCALE = float(np.sqrt(EMBED))


def _sc_body(xt_hbm, posb_hbm, tt_hbm, out_hbm,
             chan_v, pos_v, xsh, ibufs, obufs, isems, osems):
    c = lax.axis_index("c")
    s = lax.axis_index("s")
    wid = s * NUM_CORES + c

    @pl.when(s == 0)
    def _():
        pltpu.sync_copy(xt_hbm, xsh)

    plsc.subcore_barrier()

    for phase in range(PHASES):
        e = phase * NUM_WORKERS + wid
        pltpu.sync_copy(tt_hbm.at[e], chan_v)
        pltpu.sync_copy(posb_hbm.at[e], pos_v)

        def islab(lg):
            return xsh.at[pl.ds(lg * LG, LG)]

        for bi in range(2):
            pltpu.async_copy(islab(bi), ibufs[bi], isems[bi])

        def wait_out(b):
            pltpu.make_async_copy(obufs[b], out_hbm.at[pl.ds(0, LG), 0],
                                  osems[b]).wait()

        def do_slab(lg0, blg, phase):
            lg = lg0 + blg
            pltpu.make_async_copy(islab(lg), ibufs[blg], isems[blg]).wait()
            if phase > 0:
                wait_out(blg)
            else:
                @pl.when(lg0 > 0)
                def _():
                    wait_out(blg)

            for dl in range(LG):
                l = lg * LG + dl
                posv = plsc.load_gather(
                    pos_v, [jnp.full((LANES,), l, jnp.int32)])

                @plsc.parallel_loop(0, B // LANES, unroll=16)
                def _vec(j):
                    sl = pl.ds(j * LANES, LANES)
                    idxv = ibufs[blg][dl, sl]
                    g = plsc.load_gather(chan_v, [idxv])
                    obufs[blg][dl, sl] = g * _SCALE + posv

            pltpu.async_copy(obufs[blg], out_hbm.at[pl.ds(lg * LG, LG), e],
                             osems[blg])

            @pl.when(lg + 2 < NLG)
            def _():
                pltpu.async_copy(islab(lg + 2), ibufs[blg], isems[blg])

        @pl.loop(0, NLG, step=2)
        def _slabs(lg0):
            for blg in range(2):
                do_slab(lg0, blg, phase)

    for b in range(2):
        pltpu.make_async_copy(obufs[b], out_hbm.at[pl.ds(0, LG), 0],
                              osems[b]).wait()


@functools.partial(
    pl.kernel,
    out_type=jax.ShapeDtypeStruct((L, EMBED, B), jnp.float32),
    mesh=plsc.VectorSubcoreMesh(core_axis_name="c", subcore_axis_name="s"),
    compiler_params=pltpu.CompilerParams(use_tc_tiling_on_sc=True,
                                         needs_layout_passes=False),
    scratch_types=[
        pltpu.VMEM((VOCAB,), jnp.float32),
        pltpu.VMEM((L,), jnp.float32),
        pltpu.VMEM_SHARED((L, B), jnp.int32),
        [pltpu.VMEM((LG, B), jnp.int32) for _ in range(2)],
        [pltpu.VMEM((LG, B), jnp.float32) for _ in range(2)],
        [pltpu.SemaphoreType.DMA for _ in range(2)],
        [pltpu.SemaphoreType.DMA for _ in range(2)],
    ],
)
def _sc_embed(xt_hbm, posb_hbm, tt_hbm, out_hbm,
              chan_v, pos_v, xsh, ibufs, obufs, isems, osems):
    _sc_body(xt_hbm, posb_hbm, tt_hbm, out_hbm,
             chan_v, pos_v, xsh, ibufs, obufs, isems, osems)


def kernel(x, table):
    posb = jnp.asarray(_POS_B)
    out3 = _sc_embed(x.T, posb, table.T)
    return out3.transpose(2, 0, 1)
